# baseline (speedup 1.0000x reference)
import functools
import math

import jax
import jax.numpy as jnp
from jax import lax
from jax.experimental import pallas as pl
from jax.experimental.pallas import tpu as pltpu
from jax.experimental.pallas import tpu_sc as plsc

VOCAB = 32000
PAD = 0
EPS = 0.1 / (VOCAB - 2)
CONF = 0.9
GCOEF = CONF - EPS
ROW_CONST = CONF * math.log(CONF) + (VOCAB - 2) * EPS * math.log(EPS)

N_ROWS = 4096
R_TC = 2432
BR = 128
BC = 16000
GRID_R = R_TC // BR
GRID_C = VOCAB // BC


def _tc_body(tgt_ref, x_ref, out_ref):
    i = pl.program_id(0)
    j = pl.program_id(1)

    @pl.when((i == 0) & (j == 0))
    def _init():
        out_ref[0, 0] = 0.0

    tgt = tgt_ref[0, 0, :]
    mask = tgt != PAD
    xt = x_ref[...]
    rs = jnp.sum(xt, axis=1)
    part = jnp.sum(rs * jnp.where(mask, -EPS, 0.0))

    cols = lax.broadcasted_iota(jnp.int32, (BR, BC), 1)
    match = cols == (tgt - j * BC)[:, None]
    gv = jnp.sum(jnp.where(match, xt, 0.0), axis=1)
    part += jnp.sum(gv * jnp.where(mask, -GCOEF, 0.0))

    @pl.when(j == 0)
    def _col0_terms():
        x0 = xt[:, 0]
        out_ref[0, 0] += jnp.sum(jnp.where(mask, ROW_CONST + EPS * x0, 0.0))

    out_ref[0, 0] += part


def _tc_sweep(x, target3d):
    return pl.pallas_call(
        _tc_body,
        grid=(GRID_R, GRID_C),
        in_specs=[
            pl.BlockSpec((1, 1, BR), lambda i, j: (i, 0, 0)),
            pl.BlockSpec((BR, BC), lambda i, j: (i, j)),
        ],
        out_specs=pl.BlockSpec(memory_space=pltpu.SMEM),
        out_shape=jax.ShapeDtypeStruct((1, 1), jnp.float32),
        compiler_params=pltpu.CompilerParams(
            dimension_semantics=("arbitrary", "arbitrary"),
        ),
    )(target3d, x)


_INFO = plsc.get_sparse_core_info()
_NC, _NS, _L = _INFO.num_cores, _INFO.num_subcores, _INFO.num_lanes
_NW = _NC * _NS
_TROWS = (N_ROWS - R_TC) // 8
_GLO = _TROWS // _NW
_NHI = _TROWS - _GLO * _NW
_RWMAX = (_GLO + 1) * 8
_CW = 3200
_NCH = VOCAB // _CW
_CTILES = _CW // 128
_CWORDS = _CTILES * 1024


def _sc_body(x_hbm, tgt_hbm, out_hbm, tgtd_v, bufa_v, bufb_v, acc_v, rsum_v,
             asem, bsem):
    wid = lax.axis_index("s") * _NC + lax.axis_index("c")
    lane = lax.iota(jnp.int32, _L)
    ghi = jnp.minimum(wid, _NHI)
    gw = _GLO + jnp.where(wid < _NHI, 1, 0)
    trow0 = R_TC // 8 + ghi * (_GLO + 1) + (wid - ghi) * _GLO
    dbase = trow0 * 8
    qw = gw * _NCH

    pltpu.sync_copy(tgt_hbm.at[pl.ds(dbase, _GLO * 8)],
                    tgtd_v.at[pl.ds(0, _GLO * 8)])

    @pl.when(wid < _NHI)
    def _extra_tgt():
        pltpu.sync_copy(tgt_hbm.at[pl.ds(dbase + _GLO * 8, 8)],
                        tgtd_v.at[pl.ds(_GLO * 8, 8)])

    acc_v[...] = jnp.zeros((_L,), jnp.float32)

    bufs = (bufa_v, bufb_v)
    sems = (asem, bsem)

    def issue(q, par):
        qc = jnp.minimum(q, qw - 1)
        tr = trow0 + qc // _NCH
        c = qc % _NCH
        pltpu.make_async_copy(
            x_hbm.at[pl.ds(tr * 8, 8), pl.ds(c * _CW, _CW)],
            bufs[par], sems[par]).start()

    def wait(par):
        pltpu.make_async_copy(
            x_hbm.at[pl.ds(0, 8), pl.ds(0, _CW)],
            bufs[par], sems[par]).wait()

    def chunk_sums(buf):
        tpi = 5

        def tbody(k, c):
            base = k * (tpi * 128)
            for s in range(8):
                a = rsum_v[pl.ds(s * _L, _L)]
                b = jnp.zeros((_L,), jnp.float32)
                for t in range(tpi):
                    for m in range(8):
                        col = base + t * 128 + m * _L
                        if m % 2 == 0:
                            a = a + buf[s, pl.ds(col, _L)]
                        else:
                            b = b + buf[s, pl.ds(col, _L)]
                rsum_v[pl.ds(s * _L, _L)] = a + b
            return c
        lax.fori_loop(0, _CTILES // tpi, tbody, 0)

    def row_w(r_local):
        tvec = tgtd_v[pl.ds(r_local, _L)]
        tf = tvec.astype(jnp.float32)[0]
        w = jnp.where(tf != 0.0, 1.0, 0.0)
        return w, jnp.int32(tf)

    def body(i, csum):
        for q_loc in range(_NCH):
            q = i * _NCH + q_loc
            par = q_loc % 2
            c = q_loc
            g = i
            if c == 0:
                for s in range(8):
                    rsum_v[pl.ds(s * _L, _L)] = jnp.zeros((_L,), jnp.float32)
            wait(par)
            issue(q + 1, (q_loc + 1) % 2)
            buf = bufs[par]
            chunk_sums(buf)
            for s in range(8):
                w, t_r = row_w(g * 8 + s)
                tl = t_r - c * _CW
                tlc = jnp.clip(tl, 0, _CW - 1)
                ablock = buf[s, pl.ds((tlc // _L) * _L, _L)]
                inb = (tl >= 0) & (tl < _CW)
                wq = jnp.where(inb, -GCOEF * w, 0.0)
                acc_v[...] = acc_v[...] + jnp.where(
                    lane == tlc % _L, wq * ablock, 0.0)
                if c == 0:
                    x0 = buf[s, pl.ds(0, _L)][0]
                    csum += w * (EPS * x0 + ROW_CONST)
            if c == _NCH - 1:
                for s in range(8):
                    w, _ = row_w(g * 8 + s)
                    acc_v[...] = acc_v[...] + (
                        (-EPS * w) * rsum_v[pl.ds(s * _L, _L)])
        return csum

    issue(0, 0)
    csum = lax.fori_loop(0, gw, body, jnp.float32(0.0))
    wait(0)

    acc_v[...] = acc_v[...] + jnp.where(lane == 0, csum, 0.0)
    pltpu.sync_copy(acc_v, out_hbm.at[pl.ds(wid * _L, _L)])


_sc_kernel = functools.partial(
    pl.kernel,
    out_type=jax.ShapeDtypeStruct((_NW * _L,), jnp.float32),
    mesh=plsc.VectorSubcoreMesh(core_axis_name="c", subcore_axis_name="s"),
    compiler_params=pltpu.CompilerParams(use_tc_tiling_on_sc=True),
    scratch_types=[
        pltpu.VMEM((_RWMAX + _L,), jnp.int32),
        pltpu.VMEM((8, _CW), jnp.float32),
        pltpu.VMEM((8, _CW), jnp.float32),
        pltpu.VMEM((_L,), jnp.float32),
        pltpu.VMEM((8 * _L,), jnp.float32),
        pltpu.SemaphoreType.DMA,
        pltpu.SemaphoreType.DMA,
    ],
)(_sc_body)


@jax.jit
def kernel(x, target):
    target = target.astype(jnp.int32)
    target3d = target[:R_TC].reshape(GRID_R, 1, BR)
    tc_part = _tc_sweep(x, target3d)[0, 0]
    sc_part = jnp.sum(_sc_kernel(x, target))
    return tc_part + sc_part


# speedup vs baseline: 1.0518x; 1.0518x over previous
import functools
import math

import jax
import jax.numpy as jnp
from jax import lax
from jax.experimental import pallas as pl
from jax.experimental.pallas import tpu as pltpu
from jax.experimental.pallas import tpu_sc as plsc

VOCAB = 32000
PAD = 0
EPS = 0.1 / (VOCAB - 2)
CONF = 0.9
GCOEF = CONF - EPS
ROW_CONST = CONF * math.log(CONF) + (VOCAB - 2) * EPS * math.log(EPS)

N_ROWS = 4096
R_TC = 2560
BR = 256
BC = 16000
GRID_R = R_TC // BR
GRID_C = VOCAB // BC


def _tc_body(tgt_ref, x_ref, out_ref):
    i = pl.program_id(0)
    j = pl.program_id(1)

    @pl.when((i == 0) & (j == 0))
    def _init():
        out_ref[0, 0] = 0.0

    tgt = tgt_ref[0, 0, :]
    mask = tgt != PAD
    xt = x_ref[...]
    rs = jnp.sum(xt, axis=1)
    part = jnp.sum(rs * jnp.where(mask, -EPS, 0.0))

    cols = lax.broadcasted_iota(jnp.int32, (BR, BC), 1)
    match = cols == (tgt - j * BC)[:, None]
    gv = jnp.sum(jnp.where(match, xt, 0.0), axis=1)
    part += jnp.sum(gv * jnp.where(mask, -GCOEF, 0.0))

    @pl.when(j == 0)
    def _col0_terms():
        x0 = xt[:, 0]
        out_ref[0, 0] += jnp.sum(jnp.where(mask, ROW_CONST + EPS * x0, 0.0))

    out_ref[0, 0] += part


def _tc_sweep(x, target3d):
    return pl.pallas_call(
        _tc_body,
        grid=(GRID_R, GRID_C),
        in_specs=[
            pl.BlockSpec((1, 1, BR), lambda i, j: (i, 0, 0)),
            pl.BlockSpec((BR, BC), lambda i, j: (i, j)),
        ],
        out_specs=pl.BlockSpec(memory_space=pltpu.SMEM),
        out_shape=jax.ShapeDtypeStruct((1, 1), jnp.float32),
        compiler_params=pltpu.CompilerParams(
            dimension_semantics=("arbitrary", "arbitrary"),
        ),
    )(target3d, x)


_INFO = plsc.get_sparse_core_info()
_NC, _NS, _L = _INFO.num_cores, _INFO.num_subcores, _INFO.num_lanes
_NW = _NC * _NS
_TROWS = (N_ROWS - R_TC) // 8
_GLO = _TROWS // _NW
_NHI = _TROWS - _GLO * _NW
_RWMAX = (_GLO + 1) * 8
_CW = 3200
_NCH = VOCAB // _CW
_CTILES = _CW // 128
_CWORDS = _CTILES * 1024


def _sc_body(x_hbm, tgt_hbm, out_hbm, tgtd_v, bufa_v, bufb_v, acc_v, rsum_v,
             asem, bsem):
    wid = lax.axis_index("s") * _NC + lax.axis_index("c")
    lane = lax.iota(jnp.int32, _L)
    ghi = jnp.minimum(wid, _NHI)
    gw = _GLO + jnp.where(wid < _NHI, 1, 0)
    trow0 = R_TC // 8 + ghi * (_GLO + 1) + (wid - ghi) * _GLO
    dbase = trow0 * 8
    qw = gw * _NCH

    pltpu.sync_copy(tgt_hbm.at[pl.ds(dbase, _GLO * 8)],
                    tgtd_v.at[pl.ds(0, _GLO * 8)])

    @pl.when(wid < _NHI)
    def _extra_tgt():
        pltpu.sync_copy(tgt_hbm.at[pl.ds(dbase + _GLO * 8, 8)],
                        tgtd_v.at[pl.ds(_GLO * 8, 8)])

    acc_v[...] = jnp.zeros((_L,), jnp.float32)

    bufs = (bufa_v, bufb_v)
    sems = (asem, bsem)

    def issue(q, par):
        qc = jnp.minimum(q, qw - 1)
        tr = trow0 + qc // _NCH
        c = qc % _NCH
        pltpu.make_async_copy(
            x_hbm.at[pl.ds(tr * 8, 8), pl.ds(c * _CW, _CW)],
            bufs[par], sems[par]).start()

    def wait(par):
        pltpu.make_async_copy(
            x_hbm.at[pl.ds(0, 8), pl.ds(0, _CW)],
            bufs[par], sems[par]).wait()

    def chunk_sums(buf):
        tpi = 5

        def tbody(k, c):
            base = k * (tpi * 128)
            for s in range(8):
                a = rsum_v[pl.ds(s * _L, _L)]
                b = jnp.zeros((_L,), jnp.float32)
                for t in range(tpi):
                    for m in range(8):
                        col = base + t * 128 + m * _L
                        if m % 2 == 0:
                            a = a + buf[s, pl.ds(col, _L)]
                        else:
                            b = b + buf[s, pl.ds(col, _L)]
                rsum_v[pl.ds(s * _L, _L)] = a + b
            return c
        lax.fori_loop(0, _CTILES // tpi, tbody, 0)

    def row_w(r_local):
        tvec = tgtd_v[pl.ds(r_local, _L)]
        tf = tvec.astype(jnp.float32)[0]
        w = jnp.where(tf != 0.0, 1.0, 0.0)
        return w, jnp.int32(tf)

    def body(i, csum):
        for q_loc in range(_NCH):
            q = i * _NCH + q_loc
            par = q_loc % 2
            c = q_loc
            g = i
            if c == 0:
                for s in range(8):
                    rsum_v[pl.ds(s * _L, _L)] = jnp.zeros((_L,), jnp.float32)
            wait(par)
            issue(q + 1, (q_loc + 1) % 2)
            buf = bufs[par]
            chunk_sums(buf)
            for s in range(8):
                w, t_r = row_w(g * 8 + s)
                tl = t_r - c * _CW
                tlc = jnp.clip(tl, 0, _CW - 1)
                ablock = buf[s, pl.ds((tlc // _L) * _L, _L)]
                inb = (tl >= 0) & (tl < _CW)
                wq = jnp.where(inb, -GCOEF * w, 0.0)
                acc_v[...] = acc_v[...] + jnp.where(
                    lane == tlc % _L, wq * ablock, 0.0)
                if c == 0:
                    x0 = buf[s, pl.ds(0, _L)][0]
                    csum += w * (EPS * x0 + ROW_CONST)
            if c == _NCH - 1:
                for s in range(8):
                    w, _ = row_w(g * 8 + s)
                    acc_v[...] = acc_v[...] + (
                        (-EPS * w) * rsum_v[pl.ds(s * _L, _L)])
        return csum

    issue(0, 0)
    csum = lax.fori_loop(0, gw, body, jnp.float32(0.0))
    wait(0)

    acc_v[...] = acc_v[...] + jnp.where(lane == 0, csum, 0.0)
    pltpu.sync_copy(acc_v, out_hbm.at[pl.ds(wid * _L, _L)])


_sc_kernel = functools.partial(
    pl.kernel,
    out_type=jax.ShapeDtypeStruct((_NW * _L,), jnp.float32),
    mesh=plsc.VectorSubcoreMesh(core_axis_name="c", subcore_axis_name="s"),
    compiler_params=pltpu.CompilerParams(use_tc_tiling_on_sc=True),
    scratch_types=[
        pltpu.VMEM((_RWMAX + _L,), jnp.int32),
        pltpu.VMEM((8, _CW), jnp.float32),
        pltpu.VMEM((8, _CW), jnp.float32),
        pltpu.VMEM((_L,), jnp.float32),
        pltpu.VMEM((8 * _L,), jnp.float32),
        pltpu.SemaphoreType.DMA,
        pltpu.SemaphoreType.DMA,
    ],
)(_sc_body)


@jax.jit
def kernel(x, target):
    target = target.astype(jnp.int32)
    target3d = target[:R_TC].reshape(GRID_R, 1, BR)
    tc_part = _tc_sweep(x, target3d)[0, 0]
    sc_part = jnp.sum(_sc_kernel(x, target))
    return tc_part + sc_part
